# baseline (speedup 1.0000x reference)
import jax
import jax.numpy as jnp
from jax.experimental import pallas as pl
from jax.experimental.pallas import tpu as pltpu

_TM = 512


def _gate_kernel(x_ref, w_hbm, o_hbm, w_buf, acc, w_sem, o_sem):
    i = pl.program_id(0)
    n = pl.num_programs(0)
    slot = jax.lax.rem(i, 2)

    @pl.when(i == 0)
    def _load_w():
        copy = pltpu.make_async_copy(w_hbm, w_buf, w_sem)
        copy.start()
        copy.wait()

    def out_copy(step, s):
        return pltpu.make_async_copy(
            acc.at[s],
            o_hbm.at[pl.ds(step * _TM, _TM), :],
            o_sem.at[s],
        )

    @pl.when(i >= 2)
    def _free_slot():
        out_copy(i - 2, slot).wait()

    acc[slot] = jax.lax.dot_general(
        x_ref[...],
        w_buf[...],
        dimension_numbers=(((1,), (1,)), ((), ())),
        preferred_element_type=jnp.float32,
    )
    out_copy(i, slot).start()

    @pl.when(i == n - 1)
    def _drain():
        out_copy(i - 1, 1 - slot).wait()
        out_copy(i, slot).wait()


def kernel(x, W_gate):
    t, d = x.shape
    e = W_gate.shape[0]
    return pl.pallas_call(
        _gate_kernel,
        grid=(t // _TM,),
        in_specs=[
            pl.BlockSpec((_TM, d), lambda i: (i, 0)),
            pl.BlockSpec(memory_space=pl.ANY),
        ],
        out_specs=pl.BlockSpec(memory_space=pl.ANY),
        out_shape=jax.ShapeDtypeStruct((t, e), jnp.float32),
        scratch_shapes=[
            pltpu.VMEM((e, d), jnp.float32),
            pltpu.VMEM((2, _TM, e), jnp.float32),
            pltpu.SemaphoreType.DMA,
            pltpu.SemaphoreType.DMA((2,)),
        ],
        compiler_params=pltpu.CompilerParams(
            dimension_semantics=(pltpu.ARBITRARY,),
        ),
    )(x, W_gate)
